# Initial kernel scaffold; baseline (speedup 1.0000x reference)
import jax
import jax.numpy as jnp
from jax.experimental import pallas as pl


def kernel(in_feat, edge_index, W, b):
    raise NotImplementedError("write your pallas kernel here")


# speedup vs baseline: 6.0109x; 6.0109x over previous
import functools

import jax
import jax.numpy as jnp
from jax import lax
from jax.experimental import pallas as pl
from jax.experimental.pallas import tpu as pltpu
from jax.experimental.pallas import tpu_sc as plsc

_N = 10000
_E = 320000
_H = 128

_NC, _NS, _L = 2, 16, 16
_NW = _NC * _NS
_EPT = _E // _NW
_K = 80
_NCHUNK = _EPT // _K
_NRT = 624
_ZR = 16
_NB = 1024

_sc_mesh = plsc.VectorSubcoreMesh(core_axis_name="c", subcore_axis_name="s")


@functools.partial(
    pl.kernel,
    out_type=(
        jax.ShapeDtypeStruct((_NW, _N), jnp.float32),
        jax.ShapeDtypeStruct((_NW, _N), jnp.float32),
    ),
    mesh=_sc_mesh,
    scratch_types=[
        pltpu.VMEM((_EPT,), jnp.int32),
        pltpu.VMEM((_EPT,), jnp.int32),
        pltpu.VMEM((_N,), jnp.float32),
        pltpu.VMEM((_N,), jnp.float32),
    ],
    compiler_params=pltpu.CompilerParams(needs_layout_passes=False),
)
def _sc_degrees(src_hbm, dst_hbm, csrc_hbm, cdst_hbm, sidx, didx, csrc, cdst):
    wid = lax.axis_index("s") * _NC + lax.axis_index("c")
    base = wid * _EPT
    pltpu.sync_copy(src_hbm.at[pl.ds(base, _EPT)], sidx)
    pltpu.sync_copy(dst_hbm.at[pl.ds(base, _EPT)], didx)
    zeros = jnp.zeros((_L,), jnp.float32)
    ones = jnp.ones((_L,), jnp.float32)

    def zbody(i, carry):
        csrc[pl.ds(i * _L, _L)] = zeros
        cdst[pl.ds(i * _L, _L)] = zeros
        return carry

    lax.fori_loop(0, _N // _L, zbody, 0)

    def cbody(i, carry):
        s = sidx[pl.ds(i * _L, _L)]
        d = didx[pl.ds(i * _L, _L)]
        plsc.addupdate_scatter(csrc, [s], ones)
        plsc.addupdate_scatter(cdst, [d], ones)
        return carry

    lax.fori_loop(0, _EPT // _L, cbody, 0)
    pltpu.sync_copy(csrc, csrc_hbm.at[wid])
    pltpu.sync_copy(cdst, cdst_hbm.at[wid])


@functools.partial(
    pl.kernel,
    out_type=jax.ShapeDtypeStruct((_NC, _N, _H), jnp.float32),
    mesh=_sc_mesh,
    scratch_types=[
        pltpu.VMEM((_K,), jnp.int32),
        pltpu.VMEM((_K,), jnp.int32),
        pltpu.VMEM((_K, _H), jnp.float32),
        pltpu.VMEM((_ZR, _H), jnp.float32),
        pltpu.VMEM_SHARED((_N, _H), jnp.float32),
        pltpu.SemaphoreType.DMA,
    ],
    compiler_params=pltpu.CompilerParams(needs_layout_passes=False),
)
def _sc_aggregate(xs_hbm, src_hbm, dst_hbm, out_hbm,
                  sidx, didx, rows, zbuf, agg_sh, gsem):
    cid = lax.axis_index("c")
    sid = lax.axis_index("s")
    wid = sid * _NC + cid
    zeros = jnp.zeros((_L,), jnp.float32)

    def zb(i, carry):
        r = i // (_H // _L)
        col = i % (_H // _L)
        zbuf[r, pl.ds(col * _L, _L)] = zeros
        return carry

    lax.fori_loop(0, _ZR * (_H // _L), zb, 0)
    row_start = sid * _NRT
    nrows = jnp.where(sid == _NS - 1, _N - (_NS - 1) * _NRT, _NRT)
    nch = nrows // _ZR

    def zc(j, carry):
        pltpu.sync_copy(zbuf, agg_sh.at[pl.ds(row_start + j * _ZR, _ZR)])
        return carry

    lax.fori_loop(0, nch, zc, 0)
    plsc.subcore_barrier()

    ebase = wid * _EPT

    def chunk(c, carry):
        b = ebase + c * _K
        pltpu.sync_copy(src_hbm.at[pl.ds(b, _K)], sidx)
        pltpu.sync_copy(dst_hbm.at[pl.ds(b, _K)], didx)
        pltpu.async_copy(xs_hbm.at[sidx], rows, gsem).wait()
        pltpu.sync_copy(rows, agg_sh.at[didx], add=True)
        return carry

    lax.fori_loop(0, _NCHUNK, chunk, 0)
    plsc.subcore_barrier()

    def dc(j, carry):
        sl = pl.ds(row_start + j * _ZR, _ZR)
        pltpu.sync_copy(agg_sh.at[sl], zbuf)
        pltpu.sync_copy(zbuf, out_hbm.at[cid, sl])
        return carry

    lax.fori_loop(0, nch, dc, 0)


def _tc_scale_t_body(x_ref, csrc_ref, xs_ref):
    deg = jnp.sum(csrc_ref[...], axis=0)
    norm = lax.rsqrt(jnp.maximum(deg, 1.0))
    xs_ref[...] = jnp.transpose(x_ref[...]) * norm[:, None]


def _tc_out_body(aggp_ref, cdst_ref, w_ref, b_ref, out_ref):
    agg = aggp_ref[0] + aggp_ref[1]
    deg = jnp.sum(cdst_ref[...], axis=0)
    norm = lax.rsqrt(jnp.maximum(deg, 1.0))
    scaled = agg * norm[:, None]
    ot = lax.dot_general(w_ref[...], scaled, (((0,), (1,)), ((), ())),
                         preferred_element_type=jnp.float32)
    out_ref[...] = jnp.maximum(ot + jnp.transpose(b_ref[...]), 0.0)


def kernel(in_feat, edge_index, W, b):
    x_t = in_feat.reshape(_H, _N)
    src = edge_index[0]
    dst = edge_index[1]

    csrc, cdst = _sc_degrees(src, dst)

    xs = pl.pallas_call(
        _tc_scale_t_body,
        grid=(pl.cdiv(_N, _NB),),
        in_specs=[
            pl.BlockSpec((_H, _NB), lambda j: (0, j)),
            pl.BlockSpec((_NW, _NB), lambda j: (0, j)),
        ],
        out_specs=pl.BlockSpec((_NB, _H), lambda j: (j, 0)),
        out_shape=jax.ShapeDtypeStruct((_N, _H), jnp.float32),
    )(x_t, csrc)

    agg_p = _sc_aggregate(xs, src, dst)

    out_t = pl.pallas_call(
        _tc_out_body,
        grid=(pl.cdiv(_N, _NB),),
        in_specs=[
            pl.BlockSpec((_NC, _NB, _H), lambda j: (0, j, 0)),
            pl.BlockSpec((_NW, _NB), lambda j: (0, j)),
            pl.BlockSpec((_H, _H), lambda j: (0, 0)),
            pl.BlockSpec((1, _H), lambda j: (0, 0)),
        ],
        out_specs=pl.BlockSpec((_H, _NB), lambda j: (0, j)),
        out_shape=jax.ShapeDtypeStruct((_H, _N), jnp.float32),
    )(agg_p, cdst, W, b.reshape(1, _H))

    return out_t.reshape(1, _H, 1, _N)


# speedup vs baseline: 9.9180x; 1.6500x over previous
import functools

import jax
import jax.numpy as jnp
from jax import lax
from jax.experimental import pallas as pl
from jax.experimental.pallas import tpu as pltpu
from jax.experimental.pallas import tpu_sc as plsc

_N = 10000
_E = 320000
_H = 128

_NC, _NS, _L = 2, 16, 16
_NW = _NC * _NS
_EPT = _E // _NW
_K = 40
_NCHUNK = _EPT // _K
_NRT = 624
_ZR = 16
_NB = 1024

_sc_mesh = plsc.VectorSubcoreMesh(core_axis_name="c", subcore_axis_name="s")


@functools.partial(
    pl.kernel,
    out_type=(
        jax.ShapeDtypeStruct((_NW, _N), jnp.float32),
        jax.ShapeDtypeStruct((_NW, _N), jnp.float32),
    ),
    mesh=_sc_mesh,
    scratch_types=[
        pltpu.VMEM((_EPT,), jnp.int32),
        pltpu.VMEM((_EPT,), jnp.int32),
        pltpu.VMEM((_N,), jnp.float32),
        pltpu.VMEM((_N,), jnp.float32),
    ],
    compiler_params=pltpu.CompilerParams(needs_layout_passes=False),
)
def _sc_degrees(src_hbm, dst_hbm, csrc_hbm, cdst_hbm, sidx, didx, csrc, cdst):
    wid = lax.axis_index("s") * _NC + lax.axis_index("c")
    base = wid * _EPT
    pltpu.sync_copy(src_hbm.at[pl.ds(base, _EPT)], sidx)
    pltpu.sync_copy(dst_hbm.at[pl.ds(base, _EPT)], didx)
    zeros = jnp.zeros((_L,), jnp.float32)
    ones = jnp.ones((_L,), jnp.float32)

    def zbody(i, carry):
        csrc[pl.ds(i * _L, _L)] = zeros
        cdst[pl.ds(i * _L, _L)] = zeros
        return carry

    lax.fori_loop(0, _N // _L, zbody, 0)

    def cbody(i, carry):
        s = sidx[pl.ds(i * _L, _L)]
        d = didx[pl.ds(i * _L, _L)]
        plsc.addupdate_scatter(csrc, [s], ones)
        plsc.addupdate_scatter(cdst, [d], ones)
        return carry

    lax.fori_loop(0, _EPT // _L, cbody, 0)
    pltpu.sync_copy(csrc, csrc_hbm.at[wid])
    pltpu.sync_copy(cdst, cdst_hbm.at[wid])


_NBUF = 5
_NGRP = _NCHUNK // _NBUF


@functools.partial(
    pl.kernel,
    out_type=jax.ShapeDtypeStruct((_NC, _N, _H), jnp.float32),
    mesh=_sc_mesh,
    scratch_types=[
        pltpu.VMEM((_EPT,), jnp.int32),
        pltpu.VMEM((_NBUF, _K), jnp.int32),
        pltpu.VMEM((_NBUF, _K, _H), jnp.float32),
        pltpu.VMEM((_ZR, _H), jnp.float32),
        pltpu.VMEM_SHARED((_N, _H), jnp.float32),
        pltpu.SemaphoreType.DMA((_NBUF,)),
        pltpu.SemaphoreType.DMA((_NBUF,)),
    ],
    compiler_params=pltpu.CompilerParams(needs_layout_passes=False),
)
def _sc_aggregate(xs_hbm, src_hbm, dst_hbm, out_hbm,
                  sidx, didx, rows, zbuf, agg_sh, isem, gsem):
    cid = lax.axis_index("c")
    sid = lax.axis_index("s")
    wid = sid * _NC + cid
    zeros = jnp.zeros((_L,), jnp.float32)
    ebase = wid * _EPT

    sload = pltpu.async_copy(src_hbm.at[pl.ds(ebase, _EPT)], sidx,
                             gsem.at[_NBUF - 1])
    for b in range(_NBUF - 1):
        pltpu.async_copy(dst_hbm.at[pl.ds(ebase + b * _K, _K)],
                         didx.at[b], isem.at[b])

    def zb(i, carry):
        r = i // (_H // _L)
        col = i % (_H // _L)
        zbuf[r, pl.ds(col * _L, _L)] = zeros
        return carry

    lax.fori_loop(0, _ZR * (_H // _L), zb, 0)
    row_start = sid * _NRT
    nrows = jnp.where(sid == _NS - 1, _N - (_NS - 1) * _NRT, _NRT)
    nch = nrows // _ZR

    def zc(j, carry):
        pltpu.sync_copy(zbuf, agg_sh.at[pl.ds(row_start + j * _ZR, _ZR)])
        return carry

    lax.fori_loop(0, nch, zc, 0)
    sload.wait()
    plsc.subcore_barrier()

    def start_gather(c, b):
        idx = sidx.at[pl.ds(c * _K, _K)]
        return pltpu.async_copy(xs_hbm.at[idx], rows.at[b], gsem.at[b])

    def wait_gather(c, b):
        pltpu.make_async_copy(xs_hbm.at[sidx.at[pl.ds(c * _K, _K)]],
                              rows.at[b], gsem.at[b]).wait()

    def scatter(b):
        pltpu.sync_copy(rows.at[b], agg_sh.at[didx.at[b]], add=True)

    def prefetch(c, b):
        pltpu.async_copy(dst_hbm.at[pl.ds(ebase + c * _K, _K)],
                         didx.at[b], isem.at[b])

    def wait_idx(c, b):
        pltpu.make_async_copy(dst_hbm.at[pl.ds(ebase + c * _K, _K)],
                              didx.at[b], isem.at[b]).wait()

    wait_idx(0, 0)
    start_gather(0, 0)
    prefetch(_NBUF - 1, _NBUF - 1)
    for b in range(1, _NBUF):
        wait_idx(b, b)
        start_gather(b, b)
        wait_gather(b - 1, b - 1)
        scatter(b - 1)
        prefetch(b + _NBUF - 1, b - 1)

    def grp(g, carry):
        c0 = g * _NBUF
        for b in range(_NBUF):
            c = c0 + b
            pb = (b - 1) % _NBUF
            wait_idx(c, b)
            start_gather(c, b)
            wait_gather(c - 1, pb)
            scatter(pb)

            @pl.when(c + _NBUF - 1 < _NCHUNK)
            def _():
                prefetch(c + _NBUF - 1, pb)

        return carry

    lax.fori_loop(1, _NGRP, grp, 0)
    wait_gather(_NCHUNK - 1, _NBUF - 1)
    scatter(_NBUF - 1)
    plsc.subcore_barrier()

    def dc(j, carry):
        sl = pl.ds(row_start + j * _ZR, _ZR)
        pltpu.sync_copy(agg_sh.at[sl], zbuf)
        pltpu.sync_copy(zbuf, out_hbm.at[cid, sl])
        return carry

    lax.fori_loop(0, nch, dc, 0)


def _tc_scale_t_body(x_ref, csrc_ref, xs_ref):
    deg = jnp.sum(csrc_ref[...], axis=0)
    norm = lax.rsqrt(jnp.maximum(deg, 1.0))
    xs_ref[...] = jnp.transpose(x_ref[...]) * norm[:, None]


def _tc_out_body(aggp_ref, cdst_ref, w_ref, b_ref, out_ref):
    agg = aggp_ref[0] + aggp_ref[1]
    deg = jnp.sum(cdst_ref[...], axis=0)
    norm = lax.rsqrt(jnp.maximum(deg, 1.0))
    scaled = agg * norm[:, None]
    ot = lax.dot_general(w_ref[...], scaled, (((0,), (1,)), ((), ())),
                         preferred_element_type=jnp.float32)
    out_ref[...] = jnp.maximum(ot + jnp.transpose(b_ref[...]), 0.0)


def kernel(in_feat, edge_index, W, b):
    x_t = in_feat.reshape(_H, _N)
    src = edge_index[0]
    dst = edge_index[1]

    csrc, cdst = _sc_degrees(src, dst)

    xs = pl.pallas_call(
        _tc_scale_t_body,
        grid=(pl.cdiv(_N, _NB),),
        in_specs=[
            pl.BlockSpec((_H, _NB), lambda j: (0, j)),
            pl.BlockSpec((_NW, _NB), lambda j: (0, j)),
        ],
        out_specs=pl.BlockSpec((_NB, _H), lambda j: (j, 0)),
        out_shape=jax.ShapeDtypeStruct((_N, _H), jnp.float32),
    )(x_t, csrc)

    agg_p = _sc_aggregate(xs, src, dst)

    out_t = pl.pallas_call(
        _tc_out_body,
        grid=(pl.cdiv(_N, _NB),),
        in_specs=[
            pl.BlockSpec((_NC, _NB, _H), lambda j: (0, j, 0)),
            pl.BlockSpec((_NW, _NB), lambda j: (0, j)),
            pl.BlockSpec((_H, _H), lambda j: (0, 0)),
            pl.BlockSpec((1, _H), lambda j: (0, 0)),
        ],
        out_specs=pl.BlockSpec((_H, _NB), lambda j: (0, j)),
        out_shape=jax.ShapeDtypeStruct((_H, _N), jnp.float32),
    )(agg_p, cdst, W, b.reshape(1, _H))

    return out_t.reshape(1, _H, 1, _N)


# speedup vs baseline: 10.1508x; 1.0235x over previous
import functools

import jax
import jax.numpy as jnp
from jax import lax
from jax.experimental import pallas as pl
from jax.experimental.pallas import tpu as pltpu
from jax.experimental.pallas import tpu_sc as plsc

_N = 10000
_E = 320000
_H = 128

_NC, _NS, _L = 2, 16, 16
_NW = _NC * _NS
_EPT = _E // _NW
_K = 40
_NCHUNK = _EPT // _K
_NRT = 624
_ZR = 16
_NB = 1024

_sc_mesh = plsc.VectorSubcoreMesh(core_axis_name="c", subcore_axis_name="s")


@functools.partial(
    pl.kernel,
    out_type=(
        jax.ShapeDtypeStruct((_NW, _N), jnp.float32),
        jax.ShapeDtypeStruct((_NW, _N), jnp.float32),
    ),
    mesh=_sc_mesh,
    scratch_types=[
        pltpu.VMEM((_EPT,), jnp.int32),
        pltpu.VMEM((_EPT,), jnp.int32),
        pltpu.VMEM((_N,), jnp.float32),
        pltpu.VMEM((_N,), jnp.float32),
    ],
    compiler_params=pltpu.CompilerParams(needs_layout_passes=False),
)
def _sc_degrees(src_hbm, dst_hbm, csrc_hbm, cdst_hbm, sidx, didx, csrc, cdst):
    wid = lax.axis_index("s") * _NC + lax.axis_index("c")
    base = wid * _EPT
    pltpu.sync_copy(src_hbm.at[pl.ds(base, _EPT)], sidx)
    pltpu.sync_copy(dst_hbm.at[pl.ds(base, _EPT)], didx)
    zeros = jnp.zeros((_L,), jnp.float32)
    ones = jnp.ones((_L,), jnp.float32)

    def zbody(i, carry):
        csrc[pl.ds(i * _L, _L)] = zeros
        cdst[pl.ds(i * _L, _L)] = zeros
        return carry

    lax.fori_loop(0, _N // _L, zbody, 0)

    def cbody(i, carry):
        s = sidx[pl.ds(i * _L, _L)]
        d = didx[pl.ds(i * _L, _L)]
        plsc.addupdate_scatter(csrc, [s], ones)
        plsc.addupdate_scatter(cdst, [d], ones)
        return carry

    lax.fori_loop(0, _EPT // _L, cbody, 0)
    pltpu.sync_copy(csrc, csrc_hbm.at[wid])
    pltpu.sync_copy(cdst, cdst_hbm.at[wid])


_NBUF = 5
_NGRP = _NCHUNK // _NBUF


@functools.partial(
    pl.kernel,
    out_type=jax.ShapeDtypeStruct((_NC, _N, _H), jnp.float32),
    mesh=_sc_mesh,
    scratch_types=[
        pltpu.VMEM((_EPT,), jnp.int32),
        pltpu.VMEM((_NBUF, _K), jnp.int32),
        pltpu.VMEM((_NBUF, _K, _H), jnp.float32),
        pltpu.VMEM((_ZR, _H), jnp.float32),
        pltpu.VMEM_SHARED((_N, _H), jnp.float32),
        pltpu.SemaphoreType.DMA((_NBUF,)),
        pltpu.SemaphoreType.DMA((_NBUF,)),
    ],
    compiler_params=pltpu.CompilerParams(needs_layout_passes=False),
)
def _sc_aggregate(xs_hbm, src_hbm, dst_hbm, out_hbm,
                  sidx, didx, rows, zbuf, agg_sh, isem, gsem):
    cid = lax.axis_index("c")
    sid = lax.axis_index("s")
    wid = sid * _NC + cid
    zeros = jnp.zeros((_L,), jnp.float32)
    ebase = wid * _EPT

    sload = pltpu.async_copy(src_hbm.at[pl.ds(ebase, _EPT)], sidx,
                             gsem.at[_NBUF - 1])
    for b in range(_NBUF - 1):
        pltpu.async_copy(dst_hbm.at[pl.ds(ebase + b * _K, _K)],
                         didx.at[b], isem.at[b])

    def zb(i, carry):
        r = i // (_H // _L)
        col = i % (_H // _L)
        zbuf[r, pl.ds(col * _L, _L)] = zeros
        return carry

    lax.fori_loop(0, _ZR * (_H // _L), zb, 0)
    row_start = sid * _NRT
    nrows = jnp.where(sid == _NS - 1, _N - (_NS - 1) * _NRT, _NRT)
    nch = nrows // _ZR

    def zc(j, carry):
        pltpu.sync_copy(zbuf, agg_sh.at[pl.ds(row_start + j * _ZR, _ZR)])
        return carry

    lax.fori_loop(0, nch, zc, 0)
    sload.wait()
    plsc.subcore_barrier()

    def start_gather(c, b):
        idx = sidx.at[pl.ds(c * _K, _K)]
        return pltpu.async_copy(xs_hbm.at[idx], rows.at[b], gsem.at[b])

    def wait_gather(c, b):
        pltpu.make_async_copy(xs_hbm.at[sidx.at[pl.ds(c * _K, _K)]],
                              rows.at[b], gsem.at[b]).wait()

    def scatter(b):
        pltpu.sync_copy(rows.at[b], agg_sh.at[didx.at[b]], add=True)

    def prefetch(c, b):
        pltpu.async_copy(dst_hbm.at[pl.ds(ebase + c * _K, _K)],
                         didx.at[b], isem.at[b])

    def wait_idx(c, b):
        pltpu.make_async_copy(dst_hbm.at[pl.ds(ebase + c * _K, _K)],
                              didx.at[b], isem.at[b]).wait()

    wait_idx(0, 0)
    start_gather(0, 0)
    prefetch(_NBUF - 1, _NBUF - 1)
    for b in range(1, _NBUF):
        wait_idx(b, b)
        start_gather(b, b)
        wait_gather(b - 1, b - 1)
        scatter(b - 1)
        prefetch(b + _NBUF - 1, b - 1)

    def grp(g, carry):
        c0 = g * _NBUF
        for b in range(_NBUF):
            c = c0 + b
            pb = (b - 1) % _NBUF
            wait_idx(c, b)
            start_gather(c, b)
            wait_gather(c - 1, pb)
            scatter(pb)

            @pl.when(c + _NBUF - 1 < _NCHUNK)
            def _():
                prefetch(c + _NBUF - 1, pb)

        return carry

    lax.fori_loop(1, _NGRP, grp, 0)
    wait_gather(_NCHUNK - 1, _NBUF - 1)
    scatter(_NBUF - 1)
    plsc.subcore_barrier()

    @pl.when(sid < _NS - 1)
    def _():
        sl = pl.ds(row_start, _NRT)
        pltpu.sync_copy(agg_sh.at[sl], out_hbm.at[cid, sl])

    @pl.when(sid == _NS - 1)
    def _():
        sl = pl.ds(row_start, _N - (_NS - 1) * _NRT)
        pltpu.sync_copy(agg_sh.at[sl], out_hbm.at[cid, sl])


def _tc_scale_t_body(x_ref, csrc_ref, xs_ref):
    deg = jnp.sum(csrc_ref[...], axis=0)
    norm = lax.rsqrt(jnp.maximum(deg, 1.0))
    xs_ref[...] = jnp.transpose(x_ref[...]) * norm[:, None]


def _tc_out_body(aggp_ref, cdst_ref, w_ref, b_ref, out_ref):
    agg = aggp_ref[0] + aggp_ref[1]
    deg = jnp.sum(cdst_ref[...], axis=0)
    norm = lax.rsqrt(jnp.maximum(deg, 1.0))
    scaled = agg * norm[:, None]
    ot = lax.dot_general(w_ref[...], scaled, (((0,), (1,)), ((), ())),
                         preferred_element_type=jnp.float32)
    out_ref[...] = jnp.maximum(ot + jnp.transpose(b_ref[...]), 0.0)


def kernel(in_feat, edge_index, W, b):
    x_t = in_feat.reshape(_H, _N)
    src = edge_index[0]
    dst = edge_index[1]

    csrc, cdst = _sc_degrees(src, dst)

    xs = pl.pallas_call(
        _tc_scale_t_body,
        grid=(pl.cdiv(_N, _NB),),
        in_specs=[
            pl.BlockSpec((_H, _NB), lambda j: (0, j)),
            pl.BlockSpec((_NW, _NB), lambda j: (0, j)),
        ],
        out_specs=pl.BlockSpec((_NB, _H), lambda j: (j, 0)),
        out_shape=jax.ShapeDtypeStruct((_N, _H), jnp.float32),
    )(x_t, csrc)

    agg_p = _sc_aggregate(xs, src, dst)

    out_t = pl.pallas_call(
        _tc_out_body,
        grid=(pl.cdiv(_N, _NB),),
        in_specs=[
            pl.BlockSpec((_NC, _NB, _H), lambda j: (0, j, 0)),
            pl.BlockSpec((_NW, _NB), lambda j: (0, j)),
            pl.BlockSpec((_H, _H), lambda j: (0, 0)),
            pl.BlockSpec((1, _H), lambda j: (0, 0)),
        ],
        out_specs=pl.BlockSpec((_H, _NB), lambda j: (0, j)),
        out_shape=jax.ShapeDtypeStruct((_H, _N), jnp.float32),
    )(agg_p, cdst, W, b.reshape(1, _H))

    return out_t.reshape(1, _H, 1, _N)


# speedup vs baseline: 12.4167x; 1.2232x over previous
import functools

import jax
import jax.numpy as jnp
from jax import lax
from jax.experimental import pallas as pl
from jax.experimental.pallas import tpu as pltpu
from jax.experimental.pallas import tpu_sc as plsc

_N = 10000
_E = 320000
_H = 128

_NC, _NS, _L = 2, 16, 16
_NW = _NC * _NS
_EPT = _E // _NW
_K = 80
_NCHUNK = _EPT // _K
_NRT = 624
_ZR = 16
_NB = 1024

_sc_mesh = plsc.VectorSubcoreMesh(core_axis_name="c", subcore_axis_name="s")


@functools.partial(
    pl.kernel,
    out_type=(
        jax.ShapeDtypeStruct((_NW, _N), jnp.float32),
        jax.ShapeDtypeStruct((_NW, _N), jnp.float32),
    ),
    mesh=_sc_mesh,
    scratch_types=[
        pltpu.VMEM((_EPT,), jnp.int32),
        pltpu.VMEM((_EPT,), jnp.int32),
        pltpu.VMEM((_N,), jnp.float32),
        pltpu.VMEM((_N,), jnp.float32),
    ],
    compiler_params=pltpu.CompilerParams(needs_layout_passes=False),
)
def _sc_degrees(src_hbm, dst_hbm, csrc_hbm, cdst_hbm, sidx, didx, csrc, cdst):
    wid = lax.axis_index("s") * _NC + lax.axis_index("c")
    base = wid * _EPT
    pltpu.sync_copy(src_hbm.at[pl.ds(base, _EPT)], sidx)
    pltpu.sync_copy(dst_hbm.at[pl.ds(base, _EPT)], didx)
    zeros = jnp.zeros((_L,), jnp.float32)
    ones = jnp.ones((_L,), jnp.float32)

    def zbody(i, carry):
        csrc[pl.ds(i * _L, _L)] = zeros
        cdst[pl.ds(i * _L, _L)] = zeros
        return carry

    lax.fori_loop(0, _N // _L, zbody, 0)

    def cbody(i, carry):
        s = sidx[pl.ds(i * _L, _L)]
        d = didx[pl.ds(i * _L, _L)]
        plsc.addupdate_scatter(csrc, [s], ones)
        plsc.addupdate_scatter(cdst, [d], ones)
        return carry

    lax.fori_loop(0, _EPT // _L, cbody, 0)
    pltpu.sync_copy(csrc, csrc_hbm.at[wid])
    pltpu.sync_copy(cdst, cdst_hbm.at[wid])


_NBUF = 3
_NGRP = _NCHUNK // _NBUF
_REM = _NCHUNK - _NBUF * _NGRP


@functools.partial(
    pl.kernel,
    out_type=jax.ShapeDtypeStruct((_NC, _N, _H), jnp.float32),
    mesh=_sc_mesh,
    scratch_types=[
        pltpu.VMEM((_EPT,), jnp.int32),
        pltpu.VMEM((_NBUF, _K), jnp.int32),
        pltpu.VMEM((_NBUF, _K, _H), jnp.float32),
        pltpu.VMEM((_ZR, _H), jnp.float32),
        pltpu.VMEM_SHARED((_N, _H), jnp.float32),
        pltpu.SemaphoreType.DMA((_NBUF,)),
        pltpu.SemaphoreType.DMA((_NBUF,)),
    ],
    compiler_params=pltpu.CompilerParams(needs_layout_passes=False),
)
def _sc_aggregate(xs_hbm, src_hbm, dst_hbm, out_hbm,
                  sidx, didx, rows, zbuf, agg_sh, isem, gsem):
    cid = lax.axis_index("c")
    sid = lax.axis_index("s")
    wid = sid * _NC + cid
    zeros = jnp.zeros((_L,), jnp.float32)
    ebase = wid * _EPT

    sload = pltpu.async_copy(src_hbm.at[pl.ds(ebase, _EPT)], sidx,
                             gsem.at[_NBUF - 1])
    for b in range(_NBUF - 1):
        pltpu.async_copy(dst_hbm.at[pl.ds(ebase + b * _K, _K)],
                         didx.at[b], isem.at[b])

    def zb(i, carry):
        r = i // (_H // _L)
        col = i % (_H // _L)
        zbuf[r, pl.ds(col * _L, _L)] = zeros
        return carry

    lax.fori_loop(0, _ZR * (_H // _L), zb, 0)
    row_start = sid * _NRT
    nrows = jnp.where(sid == _NS - 1, _N - (_NS - 1) * _NRT, _NRT)
    nch = nrows // _ZR

    def zc(j, carry):
        pltpu.sync_copy(zbuf, agg_sh.at[pl.ds(row_start + j * _ZR, _ZR)])
        return carry

    lax.fori_loop(0, nch, zc, 0)
    sload.wait()
    plsc.subcore_barrier()

    def start_gather(c, b):
        idx = sidx.at[pl.ds(c * _K, _K)]
        return pltpu.async_copy(xs_hbm.at[idx], rows.at[b], gsem.at[b])

    def wait_gather(c, b):
        pltpu.make_async_copy(xs_hbm.at[sidx.at[pl.ds(c * _K, _K)]],
                              rows.at[b], gsem.at[b]).wait()

    def scatter(b):
        pltpu.sync_copy(rows.at[b], agg_sh.at[didx.at[b]], add=True)

    def prefetch(c, b):
        pltpu.async_copy(dst_hbm.at[pl.ds(ebase + c * _K, _K)],
                         didx.at[b], isem.at[b])

    def wait_idx(c, b):
        pltpu.make_async_copy(dst_hbm.at[pl.ds(ebase + c * _K, _K)],
                              didx.at[b], isem.at[b]).wait()

    wait_idx(0, 0)
    start_gather(0, 0)
    prefetch(_NBUF - 1, _NBUF - 1)
    for b in range(1, _NBUF):
        wait_idx(b, b)
        start_gather(b, b)
        wait_gather(b - 1, b - 1)
        scatter(b - 1)
        prefetch(b + _NBUF - 1, b - 1)

    def grp(g, carry):
        c0 = g * _NBUF
        for b in range(_NBUF):
            c = c0 + b
            pb = (b - 1) % _NBUF
            wait_idx(c, b)
            start_gather(c, b)
            wait_gather(c - 1, pb)
            scatter(pb)

            @pl.when(c + _NBUF - 1 < _NCHUNK)
            def _():
                prefetch(c + _NBUF - 1, pb)

        return carry

    lax.fori_loop(1, _NGRP, grp, 0)
    for r in range(_REM):
        c = _NBUF * _NGRP + r
        b = c % _NBUF
        wait_idx(c, b)
        start_gather(c, b)
        wait_gather(c - 1, (b - 1) % _NBUF)
        scatter((b - 1) % _NBUF)
    wait_gather(_NCHUNK - 1, (_NCHUNK - 1) % _NBUF)
    scatter((_NCHUNK - 1) % _NBUF)
    plsc.subcore_barrier()

    @pl.when(sid < _NS - 1)
    def _():
        sl = pl.ds(row_start, _NRT)
        pltpu.sync_copy(agg_sh.at[sl], out_hbm.at[cid, sl])

    @pl.when(sid == _NS - 1)
    def _():
        sl = pl.ds(row_start, _N - (_NS - 1) * _NRT)
        pltpu.sync_copy(agg_sh.at[sl], out_hbm.at[cid, sl])


def _tc_scale_t_body(x_ref, csrc_ref, xs_ref):
    deg = jnp.sum(csrc_ref[...], axis=0)
    norm = lax.rsqrt(jnp.maximum(deg, 1.0))
    xs_ref[...] = jnp.transpose(x_ref[...]) * norm[:, None]


def _tc_out_body(aggp_ref, cdst_ref, w_ref, b_ref, out_ref):
    agg = aggp_ref[0] + aggp_ref[1]
    deg = jnp.sum(cdst_ref[...], axis=0)
    norm = lax.rsqrt(jnp.maximum(deg, 1.0))
    scaled = agg * norm[:, None]
    ot = lax.dot_general(w_ref[...], scaled, (((0,), (1,)), ((), ())),
                         preferred_element_type=jnp.float32)
    out_ref[...] = jnp.maximum(ot + jnp.transpose(b_ref[...]), 0.0)


def kernel(in_feat, edge_index, W, b):
    x_t = in_feat.reshape(_H, _N)
    src = edge_index[0]
    dst = edge_index[1]

    csrc, cdst = _sc_degrees(src, dst)

    xs = pl.pallas_call(
        _tc_scale_t_body,
        grid=(pl.cdiv(_N, _NB),),
        in_specs=[
            pl.BlockSpec((_H, _NB), lambda j: (0, j)),
            pl.BlockSpec((_NW, _NB), lambda j: (0, j)),
        ],
        out_specs=pl.BlockSpec((_NB, _H), lambda j: (j, 0)),
        out_shape=jax.ShapeDtypeStruct((_N, _H), jnp.float32),
    )(x_t, csrc)

    agg_p = _sc_aggregate(xs, src, dst)

    out_t = pl.pallas_call(
        _tc_out_body,
        grid=(pl.cdiv(_N, _NB),),
        in_specs=[
            pl.BlockSpec((_NC, _NB, _H), lambda j: (0, j, 0)),
            pl.BlockSpec((_NW, _NB), lambda j: (0, j)),
            pl.BlockSpec((_H, _H), lambda j: (0, 0)),
            pl.BlockSpec((1, _H), lambda j: (0, 0)),
        ],
        out_specs=pl.BlockSpec((_H, _NB), lambda j: (0, j)),
        out_shape=jax.ShapeDtypeStruct((_H, _N), jnp.float32),
    )(agg_p, cdst, W, b.reshape(1, _H))

    return out_t.reshape(1, _H, 1, _N)


# speedup vs baseline: 12.5654x; 1.0120x over previous
import functools

import jax
import jax.numpy as jnp
from jax import lax
from jax.experimental import pallas as pl
from jax.experimental.pallas import tpu as pltpu
from jax.experimental.pallas import tpu_sc as plsc

_N = 10000
_E = 320000
_H = 128

_NC, _NS, _L = 2, 16, 16
_NW = _NC * _NS
_EPT = _E // _NW
_K = 80
_NCHUNK = _EPT // _K
_NRT = 624
_ZR = 16
_NB = 1024

_sc_mesh = plsc.VectorSubcoreMesh(core_axis_name="c", subcore_axis_name="s")


@functools.partial(
    pl.kernel,
    out_type=(
        jax.ShapeDtypeStruct((_NW, _N), jnp.float32),
        jax.ShapeDtypeStruct((_NW, _N), jnp.float32),
    ),
    mesh=_sc_mesh,
    scratch_types=[
        pltpu.VMEM((_EPT,), jnp.int32),
        pltpu.VMEM((_EPT,), jnp.int32),
        pltpu.VMEM((_N,), jnp.float32),
        pltpu.VMEM((_N,), jnp.float32),
    ],
    compiler_params=pltpu.CompilerParams(needs_layout_passes=False),
)
def _sc_degrees(src_hbm, dst_hbm, csrc_hbm, cdst_hbm, sidx, didx, csrc, cdst):
    wid = lax.axis_index("s") * _NC + lax.axis_index("c")
    base = wid * _EPT
    pltpu.sync_copy(src_hbm.at[pl.ds(base, _EPT)], sidx)
    pltpu.sync_copy(dst_hbm.at[pl.ds(base, _EPT)], didx)
    zeros = jnp.zeros((_L,), jnp.float32)
    ones = jnp.ones((_L,), jnp.float32)

    def zbody(i, carry):
        csrc[pl.ds(i * _L, _L)] = zeros
        cdst[pl.ds(i * _L, _L)] = zeros
        return carry

    lax.fori_loop(0, _N // _L, zbody, 0)

    def cbody(i, carry):
        s = sidx[pl.ds(i * _L, _L)]
        d = didx[pl.ds(i * _L, _L)]
        plsc.addupdate_scatter(csrc, [s], ones)
        plsc.addupdate_scatter(cdst, [d], ones)
        return carry

    lax.fori_loop(0, _EPT // _L, cbody, 0)
    pltpu.sync_copy(csrc, csrc_hbm.at[wid])
    pltpu.sync_copy(cdst, cdst_hbm.at[wid])


_NBUF = 3
_NGRP = _NCHUNK // _NBUF
_REM = _NCHUNK - _NBUF * _NGRP


@functools.partial(
    pl.kernel,
    out_type=jax.ShapeDtypeStruct((_NC, _N, _H), jnp.float32),
    mesh=_sc_mesh,
    scratch_types=[
        pltpu.VMEM((_EPT,), jnp.int32),
        pltpu.VMEM((_NBUF, _K), jnp.int32),
        pltpu.VMEM((_NBUF, _K, _H), jnp.float32),
        pltpu.VMEM((_ZR, _H), jnp.float32),
        pltpu.VMEM_SHARED((_N, _H), jnp.float32),
        pltpu.SemaphoreType.DMA((_NBUF,)),
        pltpu.SemaphoreType.DMA((_NBUF,)),
    ],
    compiler_params=pltpu.CompilerParams(needs_layout_passes=False),
)
def _sc_aggregate(xs_hbm, src_hbm, dst_hbm, out_hbm,
                  sidx, didx, rows, zbuf, agg_sh, isem, gsem):
    cid = lax.axis_index("c")
    sid = lax.axis_index("s")
    wid = sid * _NC + cid
    zeros = jnp.zeros((_L,), jnp.float32)
    ebase = wid * _EPT

    sload = pltpu.async_copy(src_hbm.at[pl.ds(ebase, _EPT)], sidx,
                             gsem.at[_NBUF - 1])
    for b in range(_NBUF - 1):
        pltpu.async_copy(dst_hbm.at[pl.ds(ebase + b * _K, _K)],
                         didx.at[b], isem.at[b])

    def zb(i, carry):
        r = i // (_H // _L)
        col = i % (_H // _L)
        zbuf[r, pl.ds(col * _L, _L)] = zeros
        return carry

    lax.fori_loop(0, _ZR * (_H // _L), zb, 0)
    row_start = sid * _NRT
    nrows = jnp.where(sid == _NS - 1, _N - (_NS - 1) * _NRT, _NRT)
    nch = nrows // _ZR

    def zc(j, carry):
        pltpu.sync_copy(zbuf, agg_sh.at[pl.ds(row_start + j * _ZR, _ZR)])
        return carry

    lax.fori_loop(0, nch, zc, 0)
    sload.wait()
    plsc.subcore_barrier()

    def start_gather(c, b):
        idx = sidx.at[pl.ds(c * _K, _K)]
        return pltpu.async_copy(xs_hbm.at[idx], rows.at[b], gsem.at[b])

    def wait_gather(c, b):
        pltpu.make_async_copy(xs_hbm.at[sidx.at[pl.ds(c * _K, _K)]],
                              rows.at[b], gsem.at[b]).wait()

    def scatter(b):
        pltpu.sync_copy(rows.at[b], agg_sh.at[didx.at[b]], add=True)

    def prefetch(c, b):
        pltpu.async_copy(dst_hbm.at[pl.ds(ebase + c * _K, _K)],
                         didx.at[b], isem.at[b])

    def wait_idx(c, b):
        pltpu.make_async_copy(dst_hbm.at[pl.ds(ebase + c * _K, _K)],
                              didx.at[b], isem.at[b]).wait()

    wait_idx(0, 0)
    start_gather(0, 0)
    prefetch(_NBUF - 1, _NBUF - 1)
    for b in range(1, _NBUF):
        wait_idx(b, b)
        start_gather(b, b)
        wait_gather(b - 1, b - 1)
        scatter(b - 1)
        prefetch(b + _NBUF - 1, b - 1)

    def grp(g, carry):
        c0 = g * _NBUF
        for b in range(_NBUF):
            c = c0 + b
            pb = (b - 1) % _NBUF
            wait_idx(c, b)
            start_gather(c, b)
            wait_gather(c - 1, pb)
            scatter(pb)

            @pl.when(c + _NBUF - 1 < _NCHUNK)
            def _():
                prefetch(c + _NBUF - 1, pb)

        return carry

    lax.fori_loop(1, _NGRP, grp, 0)
    for r in range(_REM):
        c = _NBUF * _NGRP + r
        b = c % _NBUF
        wait_idx(c, b)
        start_gather(c, b)
        wait_gather(c - 1, (b - 1) % _NBUF)
        scatter((b - 1) % _NBUF)
    wait_gather(_NCHUNK - 1, (_NCHUNK - 1) % _NBUF)
    scatter((_NCHUNK - 1) % _NBUF)
    plsc.subcore_barrier()

    @pl.when(sid < _NS - 1)
    def _():
        sl = pl.ds(row_start, _NRT)
        pltpu.sync_copy(agg_sh.at[sl], out_hbm.at[cid, sl])

    @pl.when(sid == _NS - 1)
    def _():
        sl = pl.ds(row_start, _N - (_NS - 1) * _NRT)
        pltpu.sync_copy(agg_sh.at[sl], out_hbm.at[cid, sl])


def _tc_split_body(e_ref, s_ref, d_ref):
    s_ref[...] = e_ref[0]
    d_ref[...] = e_ref[1]


def _tc_scale_t_body(x_ref, csrc_ref, xs_ref):
    deg = jnp.sum(csrc_ref[...], axis=0)
    norm = lax.rsqrt(jnp.maximum(deg, 1.0))
    xs_ref[...] = jnp.transpose(x_ref[0, :, 0, :]) * norm[:, None]


def _tc_out_body(aggp_ref, cdst_ref, w_ref, b_ref, out_ref):
    agg = aggp_ref[0] + aggp_ref[1]
    deg = jnp.sum(cdst_ref[...], axis=0)
    norm = lax.rsqrt(jnp.maximum(deg, 1.0))
    scaled = agg * norm[:, None]
    ot = lax.dot_general(w_ref[...], scaled, (((0,), (1,)), ((), ())),
                         preferred_element_type=jnp.float32)
    out_ref[0, :, 0, :] = jnp.maximum(ot + jnp.transpose(b_ref[...]), 0.0)


def kernel(in_feat, edge_index, W, b):
    eb = 32768
    src, dst = pl.pallas_call(
        _tc_split_body,
        grid=(pl.cdiv(_E, eb),),
        in_specs=[pl.BlockSpec((2, eb), lambda j: (0, j))],
        out_specs=[pl.BlockSpec((eb,), lambda j: (j,)),
                   pl.BlockSpec((eb,), lambda j: (j,))],
        out_shape=(jax.ShapeDtypeStruct((_E,), jnp.int32),
                   jax.ShapeDtypeStruct((_E,), jnp.int32)),
    )(edge_index)

    csrc, cdst = _sc_degrees(src, dst)

    xs = pl.pallas_call(
        _tc_scale_t_body,
        grid=(pl.cdiv(_N, _NB),),
        in_specs=[
            pl.BlockSpec((1, _H, 1, _NB), lambda j: (0, 0, 0, j)),
            pl.BlockSpec((_NW, _NB), lambda j: (0, j)),
        ],
        out_specs=pl.BlockSpec((_NB, _H), lambda j: (j, 0)),
        out_shape=jax.ShapeDtypeStruct((_N, _H), jnp.float32),
    )(in_feat, csrc)

    agg_p = _sc_aggregate(xs, src, dst)

    return pl.pallas_call(
        _tc_out_body,
        grid=(pl.cdiv(_N, _NB),),
        in_specs=[
            pl.BlockSpec((_NC, _NB, _H), lambda j: (0, j, 0)),
            pl.BlockSpec((_NW, _NB), lambda j: (0, j)),
            pl.BlockSpec((_H, _H), lambda j: (0, 0)),
            pl.BlockSpec((1, _H), lambda j: (0, 0)),
        ],
        out_specs=pl.BlockSpec((1, _H, 1, _NB), lambda j: (0, 0, 0, j)),
        out_shape=jax.ShapeDtypeStruct((1, _H, 1, _N), jnp.float32),
    )(agg_p, cdst, W, b.reshape(1, _H))


# speedup vs baseline: 13.0674x; 1.0400x over previous
import functools

import jax
import jax.numpy as jnp
from jax import lax
from jax.experimental import pallas as pl
from jax.experimental.pallas import tpu as pltpu
from jax.experimental.pallas import tpu_sc as plsc

_N = 10000
_E = 320000
_H = 128

_NC, _NS, _L = 2, 16, 16
_NW = _NC * _NS
_EPT = _E // _NW
_K = 80
_NCHUNK = _EPT // _K
_NRT = 624
_ZR = 16
_NB = 1024

_sc_mesh = plsc.VectorSubcoreMesh(core_axis_name="c", subcore_axis_name="s")


_EPT_A = 10240
_EPT_LAST = _E - (_NW - 1) * _EPT_A


@functools.partial(
    pl.kernel,
    out_type=(
        jax.ShapeDtypeStruct((_NW, _N), jnp.float32),
        jax.ShapeDtypeStruct((_NW, _N), jnp.float32),
    ),
    mesh=_sc_mesh,
    scratch_types=[
        pltpu.VMEM((2, _EPT_A), jnp.int32),
        pltpu.VMEM((_N,), jnp.float32),
        pltpu.VMEM((_N,), jnp.float32),
    ],
    compiler_params=pltpu.CompilerParams(needs_layout_passes=False),
)
def _sc_degrees(edge_hbm, csrc_hbm, cdst_hbm, edges, csrc, cdst):
    wid = lax.axis_index("s") * _NC + lax.axis_index("c")
    base = wid * _EPT_A

    @pl.when(wid < _NW - 1)
    def _():
        pltpu.sync_copy(edge_hbm.at[:, pl.ds(base, _EPT_A)], edges)

    @pl.when(wid == _NW - 1)
    def _():
        pltpu.sync_copy(edge_hbm.at[:, pl.ds(base, _EPT_LAST)],
                        edges.at[:, pl.ds(0, _EPT_LAST)])

    zeros = jnp.zeros((_L,), jnp.float32)
    ones = jnp.ones((_L,), jnp.float32)

    def zbody(i, carry):
        csrc[pl.ds(i * _L, _L)] = zeros
        cdst[pl.ds(i * _L, _L)] = zeros
        return carry

    lax.fori_loop(0, _N // _L, zbody, 0)

    def cbody(i, carry):
        s = edges[0, pl.ds(i * _L, _L)]
        d = edges[1, pl.ds(i * _L, _L)]
        plsc.addupdate_scatter(csrc, [s], ones)
        plsc.addupdate_scatter(cdst, [d], ones)
        return carry

    nit = jnp.where(wid == _NW - 1, _EPT_LAST // _L, _EPT_A // _L)
    lax.fori_loop(0, nit, cbody, 0)
    pltpu.sync_copy(csrc, csrc_hbm.at[wid])
    pltpu.sync_copy(cdst, cdst_hbm.at[wid])


_NBUF = 3
_NGRP = _NCHUNK // _NBUF
_REM = _NCHUNK - _NBUF * _NGRP


@functools.partial(
    pl.kernel,
    out_type=jax.ShapeDtypeStruct((_NC, _N, _H), jnp.float32),
    mesh=_sc_mesh,
    scratch_types=[
        pltpu.VMEM((_EPT,), jnp.int32),
        pltpu.VMEM((_NBUF, _K), jnp.int32),
        pltpu.VMEM((_NBUF, _K, _H), jnp.float32),
        pltpu.VMEM((_ZR, _H), jnp.float32),
        pltpu.VMEM_SHARED((_N, _H), jnp.float32),
        pltpu.SemaphoreType.DMA((_NBUF,)),
        pltpu.SemaphoreType.DMA((_NBUF,)),
    ],
    compiler_params=pltpu.CompilerParams(needs_layout_passes=False),
)
def _sc_aggregate(xs_hbm, src_hbm, dst_hbm, out_hbm,
                  sidx, didx, rows, zbuf, agg_sh, isem, gsem):
    cid = lax.axis_index("c")
    sid = lax.axis_index("s")
    wid = sid * _NC + cid
    zeros = jnp.zeros((_L,), jnp.float32)
    ebase = wid * _EPT

    sload = pltpu.async_copy(src_hbm.at[pl.ds(ebase, _EPT)], sidx,
                             gsem.at[_NBUF - 1])
    for b in range(_NBUF - 1):
        pltpu.async_copy(dst_hbm.at[pl.ds(ebase + b * _K, _K)],
                         didx.at[b], isem.at[b])

    def zb(i, carry):
        r = i // (_H // _L)
        col = i % (_H // _L)
        zbuf[r, pl.ds(col * _L, _L)] = zeros
        return carry

    lax.fori_loop(0, _ZR * (_H // _L), zb, 0)
    row_start = sid * _NRT
    nrows = jnp.where(sid == _NS - 1, _N - (_NS - 1) * _NRT, _NRT)
    nch = nrows // _ZR

    def zc(j, carry):
        pltpu.sync_copy(zbuf, agg_sh.at[pl.ds(row_start + j * _ZR, _ZR)])
        return carry

    lax.fori_loop(0, nch, zc, 0)
    sload.wait()
    plsc.subcore_barrier()

    def start_gather(c, b):
        idx = sidx.at[pl.ds(c * _K, _K)]
        return pltpu.async_copy(xs_hbm.at[idx], rows.at[b], gsem.at[b])

    def wait_gather(c, b):
        pltpu.make_async_copy(xs_hbm.at[sidx.at[pl.ds(c * _K, _K)]],
                              rows.at[b], gsem.at[b]).wait()

    def scatter(b):
        pltpu.sync_copy(rows.at[b], agg_sh.at[didx.at[b]], add=True)

    def prefetch(c, b):
        pltpu.async_copy(dst_hbm.at[pl.ds(ebase + c * _K, _K)],
                         didx.at[b], isem.at[b])

    def wait_idx(c, b):
        pltpu.make_async_copy(dst_hbm.at[pl.ds(ebase + c * _K, _K)],
                              didx.at[b], isem.at[b]).wait()

    wait_idx(0, 0)
    start_gather(0, 0)
    prefetch(_NBUF - 1, _NBUF - 1)
    for b in range(1, _NBUF):
        wait_idx(b, b)
        start_gather(b, b)
        wait_gather(b - 1, b - 1)
        scatter(b - 1)
        prefetch(b + _NBUF - 1, b - 1)

    def grp(g, carry):
        c0 = g * _NBUF
        for b in range(_NBUF):
            c = c0 + b
            pb = (b - 1) % _NBUF
            wait_idx(c, b)
            start_gather(c, b)
            wait_gather(c - 1, pb)
            scatter(pb)

            @pl.when(c + _NBUF - 1 < _NCHUNK)
            def _():
                prefetch(c + _NBUF - 1, pb)

        return carry

    lax.fori_loop(1, _NGRP, grp, 0)
    for r in range(_REM):
        c = _NBUF * _NGRP + r
        b = c % _NBUF
        wait_idx(c, b)
        start_gather(c, b)
        wait_gather(c - 1, (b - 1) % _NBUF)
        scatter((b - 1) % _NBUF)
    wait_gather(_NCHUNK - 1, (_NCHUNK - 1) % _NBUF)
    scatter((_NCHUNK - 1) % _NBUF)
    plsc.subcore_barrier()

    @pl.when(sid < _NS - 1)
    def _():
        sl = pl.ds(row_start, _NRT)
        pltpu.sync_copy(agg_sh.at[sl], out_hbm.at[cid, sl])

    @pl.when(sid == _NS - 1)
    def _():
        sl = pl.ds(row_start, _N - (_NS - 1) * _NRT)
        pltpu.sync_copy(agg_sh.at[sl], out_hbm.at[cid, sl])


def _tc_split_body(e_ref, s_ref, d_ref):
    s_ref[...] = e_ref[0]
    d_ref[...] = e_ref[1]


def _tc_scale_t_body(x_ref, csrc_ref, xs_ref):
    deg = jnp.sum(csrc_ref[...], axis=0)
    norm = lax.rsqrt(jnp.maximum(deg, 1.0))
    xs_ref[...] = jnp.transpose(x_ref[0, :, 0, :]) * norm[:, None]


def _tc_out_body(aggp_ref, cdst_ref, w_ref, b_ref, out_ref):
    agg = aggp_ref[0] + aggp_ref[1]
    deg = jnp.sum(cdst_ref[...], axis=0)
    norm = lax.rsqrt(jnp.maximum(deg, 1.0))
    scaled = agg * norm[:, None]
    ot = lax.dot_general(w_ref[...], scaled, (((0,), (1,)), ((), ())),
                         preferred_element_type=jnp.float32)
    out_ref[...] = jnp.maximum(ot + jnp.transpose(b_ref[...]), 0.0)


def kernel(in_feat, edge_index, W, b):
    csrc, cdst = _sc_degrees(edge_index)

    eb = 32768
    src, dst = pl.pallas_call(
        _tc_split_body,
        grid=(pl.cdiv(_E, eb),),
        in_specs=[pl.BlockSpec((2, eb), lambda j: (0, j))],
        out_specs=[pl.BlockSpec((eb,), lambda j: (j,)),
                   pl.BlockSpec((eb,), lambda j: (j,))],
        out_shape=(jax.ShapeDtypeStruct((_E,), jnp.int32),
                   jax.ShapeDtypeStruct((_E,), jnp.int32)),
    )(edge_index)

    xs = pl.pallas_call(
        _tc_scale_t_body,
        grid=(pl.cdiv(_N, _NB),),
        in_specs=[
            pl.BlockSpec((1, _H, 1, _NB), lambda j: (0, 0, 0, j)),
            pl.BlockSpec((_NW, _NB), lambda j: (0, j)),
        ],
        out_specs=pl.BlockSpec((_NB, _H), lambda j: (j, 0)),
        out_shape=jax.ShapeDtypeStruct((_N, _H), jnp.float32),
    )(in_feat, csrc)

    agg_p = _sc_aggregate(xs, src, dst)

    out_t = pl.pallas_call(
        _tc_out_body,
        grid=(pl.cdiv(_N, _NB),),
        in_specs=[
            pl.BlockSpec((_NC, _NB, _H), lambda j: (0, j, 0)),
            pl.BlockSpec((_NW, _NB), lambda j: (0, j)),
            pl.BlockSpec((_H, _H), lambda j: (0, 0)),
            pl.BlockSpec((1, _H), lambda j: (0, 0)),
        ],
        out_specs=pl.BlockSpec((_H, _NB), lambda j: (0, j)),
        out_shape=jax.ShapeDtypeStruct((_H, _N), jnp.float32),
    )(agg_p, cdst, W, b.reshape(1, _H))

    return out_t.reshape(1, _H, 1, _N)


# speedup vs baseline: 13.3183x; 1.0192x over previous
import functools

import jax
import jax.numpy as jnp
from jax import lax
from jax.experimental import pallas as pl
from jax.experimental.pallas import tpu as pltpu
from jax.experimental.pallas import tpu_sc as plsc

_N = 10000
_E = 320000
_H = 128

_NC, _NS, _L = 2, 16, 16
_NW = _NC * _NS
_EPT = _E // _NW
_K = 80
_NCHUNK = _EPT // _K
_NRT = 624
_ZR = 16
_NB = 1024

_sc_mesh = plsc.VectorSubcoreMesh(core_axis_name="c", subcore_axis_name="s")


_EPT_A = 10240
_EPT_LAST = _E - (_NW - 1) * _EPT_A


@functools.partial(
    pl.kernel,
    out_type=(
        jax.ShapeDtypeStruct((_NW, _N), jnp.float32),
        jax.ShapeDtypeStruct((_NW, _N), jnp.float32),
    ),
    mesh=_sc_mesh,
    scratch_types=[
        pltpu.VMEM((2, _EPT_A), jnp.int32),
        pltpu.VMEM((_N,), jnp.float32),
        pltpu.VMEM((_N,), jnp.float32),
    ],
    compiler_params=pltpu.CompilerParams(needs_layout_passes=False),
)
def _sc_degrees(edge_hbm, csrc_hbm, cdst_hbm, edges, csrc, cdst):
    wid = lax.axis_index("s") * _NC + lax.axis_index("c")
    base = wid * _EPT_A

    @pl.when(wid < _NW - 1)
    def _():
        pltpu.sync_copy(edge_hbm.at[:, pl.ds(base, _EPT_A)], edges)

    @pl.when(wid == _NW - 1)
    def _():
        pltpu.sync_copy(edge_hbm.at[:, pl.ds(base, _EPT_LAST)],
                        edges.at[:, pl.ds(0, _EPT_LAST)])

    zeros = jnp.zeros((_L,), jnp.float32)
    ones = jnp.ones((_L,), jnp.float32)

    def zbody(i, carry):
        csrc[pl.ds(i * _L, _L)] = zeros
        cdst[pl.ds(i * _L, _L)] = zeros
        return carry

    lax.fori_loop(0, _N // _L, zbody, 0)

    def cbody(i, carry):
        s = edges[0, pl.ds(i * _L, _L)]
        d = edges[1, pl.ds(i * _L, _L)]
        plsc.addupdate_scatter(csrc, [s], ones)
        plsc.addupdate_scatter(cdst, [d], ones)
        return carry

    nit = jnp.where(wid == _NW - 1, _EPT_LAST // _L, _EPT_A // _L)
    lax.fori_loop(0, nit, cbody, 0)
    pltpu.sync_copy(csrc, csrc_hbm.at[wid])
    pltpu.sync_copy(cdst, cdst_hbm.at[wid])


_NBUF = 3
_NGRP = _NCHUNK // _NBUF
_REM = _NCHUNK - _NBUF * _NGRP


@functools.partial(
    pl.kernel,
    out_type=jax.ShapeDtypeStruct((_NC, _N, _H), jnp.float32),
    mesh=_sc_mesh,
    scratch_types=[
        pltpu.VMEM((_EPT,), jnp.int32),
        pltpu.VMEM((_NBUF, _K), jnp.int32),
        pltpu.VMEM((_NBUF, _K, _H), jnp.float32),
        pltpu.VMEM((_ZR, _H), jnp.float32),
        pltpu.VMEM_SHARED((_N, _H), jnp.float32),
        pltpu.SemaphoreType.DMA((_NBUF,)),
        pltpu.SemaphoreType.DMA((_NBUF,)),
    ],
    compiler_params=pltpu.CompilerParams(needs_layout_passes=False),
)
def _sc_aggregate(xs_hbm, src_hbm, dst_hbm, out_hbm,
                  sidx, didx, rows, zbuf, agg_sh, isem, gsem):
    cid = lax.axis_index("c")
    sid = lax.axis_index("s")
    wid = sid * _NC + cid
    zeros = jnp.zeros((_L,), jnp.float32)
    ebase = wid * _EPT

    sload = pltpu.async_copy(src_hbm.at[pl.ds(ebase, _EPT)], sidx,
                             gsem.at[_NBUF - 1])
    for b in range(_NBUF - 1):
        pltpu.async_copy(dst_hbm.at[pl.ds(ebase + b * _K, _K)],
                         didx.at[b], isem.at[b])

    def zb(i, carry):
        r = i // (_H // _L)
        col = i % (_H // _L)
        zbuf[r, pl.ds(col * _L, _L)] = zeros
        return carry

    lax.fori_loop(0, _ZR * (_H // _L), zb, 0)
    row_start = sid * _NRT
    nrows = jnp.where(sid == _NS - 1, _N - (_NS - 1) * _NRT, _NRT)
    nch = nrows // _ZR

    def zc(j, carry):
        pltpu.sync_copy(zbuf, agg_sh.at[pl.ds(row_start + j * _ZR, _ZR)])
        return carry

    lax.fori_loop(0, nch, zc, 0)
    sload.wait()
    plsc.subcore_barrier()

    def start_gather(c, b):
        idx = sidx.at[pl.ds(c * _K, _K)]
        return pltpu.async_copy(xs_hbm.at[idx], rows.at[b], gsem.at[b])

    def wait_gather(c, b):
        pltpu.make_async_copy(xs_hbm.at[sidx.at[pl.ds(c * _K, _K)]],
                              rows.at[b], gsem.at[b]).wait()

    def scatter(b):
        pltpu.sync_copy(rows.at[b], agg_sh.at[didx.at[b]], add=True)

    def prefetch(c, b):
        pltpu.async_copy(dst_hbm.at[pl.ds(ebase + c * _K, _K)],
                         didx.at[b], isem.at[b])

    def wait_idx(c, b):
        pltpu.make_async_copy(dst_hbm.at[pl.ds(ebase + c * _K, _K)],
                              didx.at[b], isem.at[b]).wait()

    wait_idx(0, 0)
    start_gather(0, 0)
    prefetch(_NBUF - 1, _NBUF - 1)
    for b in range(1, _NBUF):
        wait_idx(b, b)
        start_gather(b, b)
        wait_gather(b - 1, b - 1)
        scatter(b - 1)
        prefetch(b + _NBUF - 1, b - 1)

    def grp(g, carry):
        c0 = g * _NBUF
        for b in range(_NBUF):
            c = c0 + b
            pb = (b - 1) % _NBUF
            wait_idx(c, b)
            start_gather(c, b)
            wait_gather(c - 1, pb)
            scatter(pb)

            @pl.when(c + _NBUF - 1 < _NCHUNK)
            def _():
                prefetch(c + _NBUF - 1, pb)

        return carry

    lax.fori_loop(1, _NGRP, grp, 0)
    for r in range(_REM):
        c = _NBUF * _NGRP + r
        b = c % _NBUF
        wait_idx(c, b)
        start_gather(c, b)
        wait_gather(c - 1, (b - 1) % _NBUF)
        scatter((b - 1) % _NBUF)
    wait_gather(_NCHUNK - 1, (_NCHUNK - 1) % _NBUF)
    scatter((_NCHUNK - 1) % _NBUF)
    plsc.subcore_barrier()

    @pl.when(sid < _NS - 1)
    def _():
        sl = pl.ds(row_start, _NRT)
        pltpu.sync_copy(agg_sh.at[sl], out_hbm.at[cid, sl])

    @pl.when(sid == _NS - 1)
    def _():
        sl = pl.ds(row_start, _N - (_NS - 1) * _NRT)
        pltpu.sync_copy(agg_sh.at[sl], out_hbm.at[cid, sl])


def _tc_split_body(e_ref, s_ref, d_ref):
    s_ref[...] = e_ref[0]
    d_ref[...] = e_ref[1]


def _tc_scale_t_body(x_ref, csrc_ref, xs_ref):
    deg = jnp.sum(csrc_ref[...], axis=0)
    norm = lax.rsqrt(jnp.maximum(deg, 1.0))
    xs_ref[...] = jnp.transpose(x_ref[...]) * norm[:, None]


def _tc_out_body(aggp_ref, cdst_ref, w_ref, b_ref, out_ref):
    agg = aggp_ref[0] + aggp_ref[1]
    deg = jnp.sum(cdst_ref[...], axis=0)
    norm = lax.rsqrt(jnp.maximum(deg, 1.0))
    scaled = agg * norm[:, None]
    ot = lax.dot_general(w_ref[...], scaled, (((0,), (1,)), ((), ())),
                         preferred_element_type=jnp.float32)
    out_ref[...] = jnp.maximum(ot + jnp.transpose(b_ref[...]), 0.0)


def kernel(in_feat, edge_index, W, b):
    csrc, cdst = _sc_degrees(edge_index)

    eb = 32768
    src, dst = pl.pallas_call(
        _tc_split_body,
        grid=(pl.cdiv(_E, eb),),
        in_specs=[pl.BlockSpec((2, eb), lambda j: (0, j))],
        out_specs=[pl.BlockSpec((eb,), lambda j: (j,)),
                   pl.BlockSpec((eb,), lambda j: (j,))],
        out_shape=(jax.ShapeDtypeStruct((_E,), jnp.int32),
                   jax.ShapeDtypeStruct((_E,), jnp.int32)),
    )(edge_index)

    xs = pl.pallas_call(
        _tc_scale_t_body,
        grid=(pl.cdiv(_N, _NB),),
        in_specs=[
            pl.BlockSpec((_H, _NB), lambda j: (0, j)),
            pl.BlockSpec((_NW, _NB), lambda j: (0, j)),
        ],
        out_specs=pl.BlockSpec((_NB, _H), lambda j: (j, 0)),
        out_shape=jax.ShapeDtypeStruct((_N, _H), jnp.float32),
    )(in_feat.reshape(_H, _N), csrc)

    agg_p = _sc_aggregate(xs, src, dst)

    out_t = pl.pallas_call(
        _tc_out_body,
        grid=(pl.cdiv(_N, _NB),),
        in_specs=[
            pl.BlockSpec((_NC, _NB, _H), lambda j: (0, j, 0)),
            pl.BlockSpec((_NW, _NB), lambda j: (0, j)),
            pl.BlockSpec((_H, _H), lambda j: (0, 0)),
            pl.BlockSpec((1, _H), lambda j: (0, 0)),
        ],
        out_specs=pl.BlockSpec((_H, _NB), lambda j: (0, j)),
        out_shape=jax.ShapeDtypeStruct((_H, _N), jnp.float32),
    )(agg_p, cdst, W, b.reshape(1, _H))

    return out_t.reshape(1, _H, 1, _N)


# speedup vs baseline: 13.5263x; 1.0156x over previous
import functools

import jax
import jax.numpy as jnp
from jax import lax
from jax.experimental import pallas as pl
from jax.experimental.pallas import tpu as pltpu
from jax.experimental.pallas import tpu_sc as plsc

_N = 10000
_E = 320000
_H = 128

_NC, _NS, _L = 2, 16, 16
_NW = _NC * _NS
_EPT = _E // _NW
_K = 80
_NCHUNK = _EPT // _K
_NRT = 624
_ZR = 16
_NB = 1024

_sc_mesh = plsc.VectorSubcoreMesh(core_axis_name="c", subcore_axis_name="s")


_EPT_A = 10240
_EPT_LAST = _E - (_NW - 1) * _EPT_A


@functools.partial(
    pl.kernel,
    out_type=(
        jax.ShapeDtypeStruct((_NW, _N), jnp.float32),
        jax.ShapeDtypeStruct((_NW, _N), jnp.float32),
    ),
    mesh=_sc_mesh,
    scratch_types=[
        pltpu.VMEM((2, _EPT_A), jnp.int32),
        pltpu.VMEM((_N,), jnp.float32),
        pltpu.VMEM((_N,), jnp.float32),
    ],
    compiler_params=pltpu.CompilerParams(needs_layout_passes=False),
)
def _sc_degrees(edge_hbm, csrc_hbm, cdst_hbm, edges, csrc, cdst):
    wid = lax.axis_index("s") * _NC + lax.axis_index("c")
    base = wid * _EPT_A

    @pl.when(wid < _NW - 1)
    def _():
        pltpu.sync_copy(edge_hbm.at[:, pl.ds(base, _EPT_A)], edges)

    @pl.when(wid == _NW - 1)
    def _():
        pltpu.sync_copy(edge_hbm.at[:, pl.ds(base, _EPT_LAST)],
                        edges.at[:, pl.ds(0, _EPT_LAST)])

    zeros = jnp.zeros((_L,), jnp.float32)
    ones = jnp.ones((_L,), jnp.float32)

    def zbody(i, carry):
        csrc[pl.ds(i * _L, _L)] = zeros
        cdst[pl.ds(i * _L, _L)] = zeros
        return carry

    lax.fori_loop(0, _N // _L, zbody, 0)

    def cbody(i, carry):
        s = edges[0, pl.ds(i * _L, _L)]
        d = edges[1, pl.ds(i * _L, _L)]
        plsc.addupdate_scatter(csrc, [s], ones)
        plsc.addupdate_scatter(cdst, [d], ones)
        return carry

    nit = jnp.where(wid == _NW - 1, _EPT_LAST // _L, _EPT_A // _L)
    lax.fori_loop(0, nit, cbody, 0)
    pltpu.sync_copy(csrc, csrc_hbm.at[wid])
    pltpu.sync_copy(cdst, cdst_hbm.at[wid])


_RR = 4
_IR = 8
_NGRP = _NCHUNK // _IR
_REM = _NCHUNK - _IR * _NGRP


@functools.partial(
    pl.kernel,
    out_type=jax.ShapeDtypeStruct((_NC, _N, _H), jnp.float32),
    mesh=_sc_mesh,
    scratch_types=[
        pltpu.VMEM((_IR, _K), jnp.int32),
        pltpu.VMEM((_IR, _K), jnp.int32),
        pltpu.VMEM((_RR, _K, _H), jnp.float32),
        pltpu.VMEM((_ZR, _H), jnp.float32),
        pltpu.VMEM_SHARED((_N, _H), jnp.float32),
        pltpu.SemaphoreType.DMA((_IR,)),
        pltpu.SemaphoreType.DMA((_IR,)),
        pltpu.SemaphoreType.DMA((_RR,)),
        pltpu.SemaphoreType.DMA((_RR,)),
    ],
    compiler_params=pltpu.CompilerParams(needs_layout_passes=False),
)
def _sc_aggregate(xs_hbm, src_hbm, dst_hbm, out_hbm,
                  sidx, didx, rows, zbuf, agg_sh, spsem, dpsem, gsem, ssem):
    cid = lax.axis_index("c")
    sid = lax.axis_index("s")
    wid = sid * _NC + cid
    zeros = jnp.zeros((_L,), jnp.float32)
    ebase = wid * _EPT

    def pf(c, ib):
        pltpu.async_copy(src_hbm.at[pl.ds(ebase + c * _K, _K)],
                         sidx.at[ib], spsem.at[ib])
        pltpu.async_copy(dst_hbm.at[pl.ds(ebase + c * _K, _K)],
                         didx.at[ib], dpsem.at[ib])

    def wait_pf(c, ib):
        pltpu.make_async_copy(src_hbm.at[pl.ds(ebase + c * _K, _K)],
                              sidx.at[ib], spsem.at[ib]).wait()
        pltpu.make_async_copy(dst_hbm.at[pl.ds(ebase + c * _K, _K)],
                              didx.at[ib], dpsem.at[ib]).wait()

    def sg(rb, ib):
        pltpu.async_copy(xs_hbm.at[sidx.at[ib]], rows.at[rb], gsem.at[rb])

    def wg(rb, ib):
        pltpu.make_async_copy(xs_hbm.at[sidx.at[ib]], rows.at[rb],
                              gsem.at[rb]).wait()

    def ss(rb, ib):
        pltpu.async_copy(rows.at[rb], agg_sh.at[didx.at[ib]], ssem.at[rb],
                         add=True)

    def ws(rb, ib):
        pltpu.make_async_copy(rows.at[rb], agg_sh.at[didx.at[ib]],
                              ssem.at[rb]).wait()

    for c in range(_RR):
        pf(c, c)

    def zb(i, carry):
        r = i // (_H // _L)
        col = i % (_H // _L)
        zbuf[r, pl.ds(col * _L, _L)] = zeros
        return carry

    lax.fori_loop(0, _ZR * (_H // _L), zb, 0)
    row_start = sid * _NRT
    nrows = jnp.where(sid == _NS - 1, _N - (_NS - 1) * _NRT, _NRT)
    nch = nrows // _ZR

    def zc(j, carry):
        pltpu.sync_copy(zbuf, agg_sh.at[pl.ds(row_start + j * _ZR, _ZR)])
        return carry

    lax.fori_loop(0, nch, zc, 0)
    plsc.subcore_barrier()

    for c in range(_IR):
        wait_pf(c, c)
        if c >= _RR:
            ws((c - _RR) % _RR, (c - _RR) % _IR)
        sg(c % _RR, c)
        if c >= 1:
            wg((c - 1) % _RR, c - 1)
            ss((c - 1) % _RR, c - 1)
        if c + _RR < _NCHUNK:
            pf(c + _RR, (c + _RR) % _IR)

    def grp(g, carry):
        c0 = g * _IR
        for b in range(_IR):
            c = c0 + b
            rb = b % _RR
            wait_pf(c, b)
            ws(rb, (b + _RR) % _IR)
            sg(rb, b)
            wg((b - 1) % _RR, (b - 1) % _IR)
            ss((b - 1) % _RR, (b - 1) % _IR)

            @pl.when(c + _RR < _NCHUNK)
            def _():
                pf(c + _RR, (b + _RR) % _IR)

        return carry

    lax.fori_loop(1, _NGRP, grp, 0)

    for r in range(_REM):
        c = _IR * _NGRP + r
        b = c % _IR
        rb = c % _RR
        wait_pf(c, b)
        ws((c - _RR) % _RR, (c - _RR) % _IR)
        sg(rb, b)
        wg((c - 1) % _RR, (c - 1) % _IR)
        ss((c - 1) % _RR, (c - 1) % _IR)
        if c + _RR < _NCHUNK:
            pf(c + _RR, (c + _RR) % _IR)
    wg((_NCHUNK - 1) % _RR, (_NCHUNK - 1) % _IR)
    ss((_NCHUNK - 1) % _RR, (_NCHUNK - 1) % _IR)
    for c in range(_NCHUNK - _RR, _NCHUNK):
        ws(c % _RR, c % _IR)
    plsc.subcore_barrier()

    @pl.when(sid < _NS - 1)
    def _():
        sl = pl.ds(row_start, _NRT)
        pltpu.sync_copy(agg_sh.at[sl], out_hbm.at[cid, sl])

    @pl.when(sid == _NS - 1)
    def _():
        sl = pl.ds(row_start, _N - (_NS - 1) * _NRT)
        pltpu.sync_copy(agg_sh.at[sl], out_hbm.at[cid, sl])


def _tc_split_body(e_ref, s_ref, d_ref):
    s_ref[...] = e_ref[0]
    d_ref[...] = e_ref[1]


def _tc_scale_t_body(x_ref, csrc_ref, xs_ref):
    deg = jnp.sum(csrc_ref[...], axis=0)
    norm = lax.rsqrt(jnp.maximum(deg, 1.0))
    xs_ref[...] = jnp.transpose(x_ref[...]) * norm[:, None]


def _tc_out_body(aggp_ref, cdst_ref, w_ref, b_ref, out_ref):
    agg = aggp_ref[0] + aggp_ref[1]
    deg = jnp.sum(cdst_ref[...], axis=0)
    norm = lax.rsqrt(jnp.maximum(deg, 1.0))
    scaled = agg * norm[:, None]
    ot = lax.dot_general(w_ref[...], scaled, (((0,), (1,)), ((), ())),
                         preferred_element_type=jnp.float32)
    out_ref[...] = jnp.maximum(ot + jnp.transpose(b_ref[...]), 0.0)


def kernel(in_feat, edge_index, W, b):
    csrc, cdst = _sc_degrees(edge_index)

    eb = 32768
    src, dst = pl.pallas_call(
        _tc_split_body,
        grid=(pl.cdiv(_E, eb),),
        in_specs=[pl.BlockSpec((2, eb), lambda j: (0, j))],
        out_specs=[pl.BlockSpec((eb,), lambda j: (j,)),
                   pl.BlockSpec((eb,), lambda j: (j,))],
        out_shape=(jax.ShapeDtypeStruct((_E,), jnp.int32),
                   jax.ShapeDtypeStruct((_E,), jnp.int32)),
    )(edge_index)

    xs = pl.pallas_call(
        _tc_scale_t_body,
        grid=(pl.cdiv(_N, _NB),),
        in_specs=[
            pl.BlockSpec((_H, _NB), lambda j: (0, j)),
            pl.BlockSpec((_NW, _NB), lambda j: (0, j)),
        ],
        out_specs=pl.BlockSpec((_NB, _H), lambda j: (j, 0)),
        out_shape=jax.ShapeDtypeStruct((_N, _H), jnp.float32),
    )(in_feat.reshape(_H, _N), csrc)

    agg_p = _sc_aggregate(xs, src, dst)

    out_t = pl.pallas_call(
        _tc_out_body,
        grid=(pl.cdiv(_N, _NB),),
        in_specs=[
            pl.BlockSpec((_NC, _NB, _H), lambda j: (0, j, 0)),
            pl.BlockSpec((_NW, _NB), lambda j: (0, j)),
            pl.BlockSpec((_H, _H), lambda j: (0, 0)),
            pl.BlockSpec((1, _H), lambda j: (0, 0)),
        ],
        out_specs=pl.BlockSpec((_H, _NB), lambda j: (0, j)),
        out_shape=jax.ShapeDtypeStruct((_H, _N), jnp.float32),
    )(agg_p, cdst, W, b.reshape(1, _H))

    return out_t.reshape(1, _H, 1, _N)


# speedup vs baseline: 14.2681x; 1.0548x over previous
import functools

import jax
import jax.numpy as jnp
from jax import lax
from jax.experimental import pallas as pl
from jax.experimental.pallas import tpu as pltpu
from jax.experimental.pallas import tpu_sc as plsc

_N = 10000
_E = 320000
_H = 128

_NC, _NS, _L = 2, 16, 16
_NW = _NC * _NS
_EPT = _E // _NW
_K = 80
_NCHUNK = _EPT // _K
_NRT = 624
_ZR = 16
_NB = 1024

_sc_mesh = plsc.VectorSubcoreMesh(core_axis_name="c", subcore_axis_name="s")


_EPT_A = 10240
_EPT_LAST = _E - (_NW - 1) * _EPT_A


@functools.partial(
    pl.kernel,
    out_type=(
        jax.ShapeDtypeStruct((_NW, _N), jnp.float32),
        jax.ShapeDtypeStruct((_NW, _N), jnp.float32),
    ),
    mesh=_sc_mesh,
    scratch_types=[
        pltpu.VMEM((2, _EPT_A), jnp.int32),
        pltpu.VMEM((_N,), jnp.float32),
        pltpu.VMEM((_N,), jnp.float32),
    ],
    compiler_params=pltpu.CompilerParams(needs_layout_passes=False),
)
def _sc_degrees(edge_hbm, csrc_hbm, cdst_hbm, edges, csrc, cdst):
    wid = lax.axis_index("s") * _NC + lax.axis_index("c")
    base = wid * _EPT_A

    @pl.when(wid < _NW - 1)
    def _():
        pltpu.sync_copy(edge_hbm.at[:, pl.ds(base, _EPT_A)], edges)

    @pl.when(wid == _NW - 1)
    def _():
        pltpu.sync_copy(edge_hbm.at[:, pl.ds(base, _EPT_LAST)],
                        edges.at[:, pl.ds(0, _EPT_LAST)])

    zeros = jnp.zeros((_L,), jnp.float32)
    ones = jnp.ones((_L,), jnp.float32)

    def zbody(i, carry):
        csrc[pl.ds(i * _L, _L)] = zeros
        cdst[pl.ds(i * _L, _L)] = zeros
        return carry

    lax.fori_loop(0, _N // _L, zbody, 0)

    def cbody(i, carry):
        s = edges[0, pl.ds(i * _L, _L)]
        d = edges[1, pl.ds(i * _L, _L)]
        plsc.addupdate_scatter(csrc, [s], ones)
        plsc.addupdate_scatter(cdst, [d], ones)
        return carry

    nit = jnp.where(wid == _NW - 1, _EPT_LAST // _L, _EPT_A // _L)
    lax.fori_loop(0, nit, cbody, 0)
    pltpu.sync_copy(csrc, csrc_hbm.at[wid])
    pltpu.sync_copy(cdst, cdst_hbm.at[wid])


_RR = 4
_IR = 8
_NGRP = _NCHUNK // _IR
_REM = _NCHUNK - _IR * _NGRP


@functools.partial(
    pl.kernel,
    out_type=jax.ShapeDtypeStruct((_NC, _N, _H), jnp.bfloat16),
    mesh=_sc_mesh,
    scratch_types=[
        pltpu.VMEM((_IR, _K), jnp.int32),
        pltpu.VMEM((_IR, _K), jnp.int32),
        pltpu.VMEM((_RR, _K, _H), jnp.bfloat16),
        pltpu.VMEM((_ZR, _H), jnp.bfloat16),
        pltpu.VMEM_SHARED((_N, _H), jnp.bfloat16),
        pltpu.SemaphoreType.DMA((_IR,)),
        pltpu.SemaphoreType.DMA((_IR,)),
        pltpu.SemaphoreType.DMA((_RR,)),
        pltpu.SemaphoreType.DMA((_RR,)),
    ],
    compiler_params=pltpu.CompilerParams(needs_layout_passes=False,
                                         use_tc_tiling_on_sc=False),
)
def _sc_aggregate(xs_hbm, src_hbm, dst_hbm, out_hbm,
                  sidx, didx, rows, zbuf, agg_sh, spsem, dpsem, gsem, ssem):
    cid = lax.axis_index("c")
    sid = lax.axis_index("s")
    wid = sid * _NC + cid
    zeros = jnp.zeros((2 * _L,), jnp.bfloat16)
    ebase = wid * _EPT

    def pf(c, ib):
        pltpu.async_copy(src_hbm.at[pl.ds(ebase + c * _K, _K)],
                         sidx.at[ib], spsem.at[ib])
        pltpu.async_copy(dst_hbm.at[pl.ds(ebase + c * _K, _K)],
                         didx.at[ib], dpsem.at[ib])

    def wait_pf(c, ib):
        pltpu.make_async_copy(src_hbm.at[pl.ds(ebase + c * _K, _K)],
                              sidx.at[ib], spsem.at[ib]).wait()
        pltpu.make_async_copy(dst_hbm.at[pl.ds(ebase + c * _K, _K)],
                              didx.at[ib], dpsem.at[ib]).wait()

    def sg(rb, ib):
        pltpu.async_copy(xs_hbm.at[sidx.at[ib]], rows.at[rb], gsem.at[rb])

    def wg(rb, ib):
        pltpu.make_async_copy(xs_hbm.at[sidx.at[ib]], rows.at[rb],
                              gsem.at[rb]).wait()

    def ss(rb, ib):
        pltpu.async_copy(rows.at[rb], agg_sh.at[didx.at[ib]], ssem.at[rb],
                         add=True)

    def ws(rb, ib):
        pltpu.make_async_copy(rows.at[rb], agg_sh.at[didx.at[ib]],
                              ssem.at[rb]).wait()

    for c in range(_RR):
        pf(c, c)

    def zb(i, carry):
        r = i // (_H // (2 * _L))
        col = i % (_H // (2 * _L))
        zbuf[r, pl.ds(col * 2 * _L, 2 * _L)] = zeros
        return carry

    lax.fori_loop(0, _ZR * (_H // (2 * _L)), zb, 0)
    row_start = sid * _NRT
    nrows = jnp.where(sid == _NS - 1, _N - (_NS - 1) * _NRT, _NRT)
    nch = nrows // _ZR

    def zc(j, carry):
        pltpu.sync_copy(zbuf, agg_sh.at[pl.ds(row_start + j * _ZR, _ZR)])
        return carry

    lax.fori_loop(0, nch, zc, 0)
    plsc.subcore_barrier()

    for c in range(_IR):
        wait_pf(c, c)
        if c >= _RR:
            ws((c - _RR) % _RR, (c - _RR) % _IR)
        sg(c % _RR, c)
        if c >= 1:
            wg((c - 1) % _RR, c - 1)
            ss((c - 1) % _RR, c - 1)
        if c + _RR < _NCHUNK:
            pf(c + _RR, (c + _RR) % _IR)

    def grp(g, carry):
        c0 = g * _IR
        for b in range(_IR):
            c = c0 + b
            rb = b % _RR
            wait_pf(c, b)
            ws(rb, (b + _RR) % _IR)
            sg(rb, b)
            wg((b - 1) % _RR, (b - 1) % _IR)
            ss((b - 1) % _RR, (b - 1) % _IR)

            @pl.when(c + _RR < _NCHUNK)
            def _():
                pf(c + _RR, (b + _RR) % _IR)

        return carry

    lax.fori_loop(1, _NGRP, grp, 0)

    for r in range(_REM):
        c = _IR * _NGRP + r
        b = c % _IR
        rb = c % _RR
        wait_pf(c, b)
        ws((c - _RR) % _RR, (c - _RR) % _IR)
        sg(rb, b)
        wg((c - 1) % _RR, (c - 1) % _IR)
        ss((c - 1) % _RR, (c - 1) % _IR)
        if c + _RR < _NCHUNK:
            pf(c + _RR, (c + _RR) % _IR)
    wg((_NCHUNK - 1) % _RR, (_NCHUNK - 1) % _IR)
    ss((_NCHUNK - 1) % _RR, (_NCHUNK - 1) % _IR)
    for c in range(_NCHUNK - _RR, _NCHUNK):
        ws(c % _RR, c % _IR)
    plsc.subcore_barrier()

    @pl.when(sid < _NS - 1)
    def _():
        sl = pl.ds(row_start, _NRT)
        pltpu.sync_copy(agg_sh.at[sl], out_hbm.at[cid, sl])

    @pl.when(sid == _NS - 1)
    def _():
        sl = pl.ds(row_start, _N - (_NS - 1) * _NRT)
        pltpu.sync_copy(agg_sh.at[sl], out_hbm.at[cid, sl])


def _tc_split_body(e_ref, s_ref, d_ref):
    s_ref[...] = e_ref[0]
    d_ref[...] = e_ref[1]


def _tc_scale_t_body(x_ref, csrc_ref, xs_ref):
    deg = jnp.sum(csrc_ref[...], axis=0)
    norm = lax.rsqrt(jnp.maximum(deg, 1.0))
    xs_ref[...] = (jnp.transpose(x_ref[...]) * norm[:, None]).astype(jnp.bfloat16)


def _tc_out_body(aggp_ref, cdst_ref, w_ref, b_ref, out_ref):
    agg = aggp_ref[0].astype(jnp.float32) + aggp_ref[1].astype(jnp.float32)
    deg = jnp.sum(cdst_ref[...], axis=0)
    norm = lax.rsqrt(jnp.maximum(deg, 1.0))
    scaled = agg * norm[:, None]
    ot = lax.dot_general(w_ref[...], scaled, (((0,), (1,)), ((), ())),
                         preferred_element_type=jnp.float32)
    out_ref[...] = jnp.maximum(ot + jnp.transpose(b_ref[...]), 0.0)


def kernel(in_feat, edge_index, W, b):
    csrc, cdst = _sc_degrees(edge_index)

    eb = 32768
    src, dst = pl.pallas_call(
        _tc_split_body,
        grid=(pl.cdiv(_E, eb),),
        in_specs=[pl.BlockSpec((2, eb), lambda j: (0, j))],
        out_specs=[pl.BlockSpec((eb,), lambda j: (j,)),
                   pl.BlockSpec((eb,), lambda j: (j,))],
        out_shape=(jax.ShapeDtypeStruct((_E,), jnp.int32),
                   jax.ShapeDtypeStruct((_E,), jnp.int32)),
    )(edge_index)

    xs = pl.pallas_call(
        _tc_scale_t_body,
        grid=(pl.cdiv(_N, _NB),),
        in_specs=[
            pl.BlockSpec((_H, _NB), lambda j: (0, j)),
            pl.BlockSpec((_NW, _NB), lambda j: (0, j)),
        ],
        out_specs=pl.BlockSpec((_NB, _H), lambda j: (j, 0)),
        out_shape=jax.ShapeDtypeStruct((_N, _H), jnp.bfloat16),
    )(in_feat.reshape(_H, _N), csrc)

    agg_p = _sc_aggregate(xs, src, dst)

    out_t = pl.pallas_call(
        _tc_out_body,
        grid=(pl.cdiv(_N, _NB),),
        in_specs=[
            pl.BlockSpec((_NC, _NB, _H), lambda j: (0, j, 0)),
            pl.BlockSpec((_NW, _NB), lambda j: (0, j)),
            pl.BlockSpec((_H, _H), lambda j: (0, 0)),
            pl.BlockSpec((1, _H), lambda j: (0, 0)),
        ],
        out_specs=pl.BlockSpec((_H, _NB), lambda j: (0, j)),
        out_shape=jax.ShapeDtypeStruct((_H, _N), jnp.float32),
    )(agg_p, cdst, W, b.reshape(1, _H))

    return out_t.reshape(1, _H, 1, _N)
